# baseline (speedup 1.0000x reference)
import functools

import jax
import jax.numpy as jnp
from jax import lax
from jax.experimental import pallas as pl
from jax.experimental.pallas import tpu as pltpu

_XTAPS = ((1, -1), (0, 0), (1, 0), (0, 1))


def _decoder_body(x_ref, i_ref, w0_ref, b0_ref, w1_ref, b1_ref, o_ref,
                  v_ref, s1_ref, s2_ref, *, H, W):
    HW = H * W
    P = W + 1
    xv = x_ref[0].astype(jnp.bfloat16)
    iv = i_ref[0]
    lane = lax.broadcasted_iota(jnp.int32, (1, HW), 1)
    ih = lane // W
    iw = lane - W * ih
    base = (2 * ih) * (2 * W) + 2 * iw

    for s_ref in (s1_ref, s2_ref):
        s_ref[:, :P, :] = jnp.zeros((4, P, s_ref.shape[2]), jnp.bfloat16)
        s_ref[:, P + HW:, :] = jnp.zeros((4, P, s_ref.shape[2]),
                                         jnp.bfloat16)
    for py in range(2):
        for px in range(2):
            m = jnp.where(iv == base + (py * 2 * W + px), xv,
                          jnp.bfloat16(0))
            s1_ref[2 * py + px, P:P + HW, :] = jnp.transpose(m)

    col_id = lax.broadcasted_iota(jnp.int32, (HW, 1), 0) % W
    not_left = col_id != 0
    not_right = col_id != (W - 1)

    def conv_layer(s_ref, wm, bi, out_f32):
        zs = []
        for py in range(2):
            taps = []
            for a in range(3):
                oy = a - 1
                sy = (py + oy) % 2
                gy = (py + oy) // 2
                for b in range(4):
                    sx, gx = _XTAPS[b]
                    s = P + gy * W + gx
                    t = s_ref[2 * sy + sx, s:s + HW, :]
                    if gx == -1:
                        t = jnp.where(not_left, t, 0)
                    elif gx == 1:
                        t = jnp.where(not_right, t, 0)
                    taps.append(t)
            col = jnp.concatenate(taps, axis=1)
            z = jnp.dot(col, wm, preferred_element_type=jnp.float32)
            if not out_f32:
                z = z.astype(jnp.bfloat16)
            zs.append(jnp.maximum(z + bi.astype(z.dtype), 0))
        return zs

    z1 = conv_layer(s1_ref, w0_ref[...], b0_ref[...], False)
    c1 = z1[0].shape[1] // 2
    for py in range(2):
        for px in range(2):
            s2_ref[2 * py + px, P:P + HW, :] = (
                z1[py][:, px * c1:(px + 1) * c1])
    z2 = conv_layer(s2_ref, w1_ref[...], b1_ref[...], True)
    c2 = z2[0].shape[1] // 2
    for py in range(2):
        for px in range(2):
            v_ref[:, py, pl.Slice(px, W, 2), :] = (
                z2[py][:, px * c2:(px + 1) * c2].reshape(H, W, c2))
    o_ref[0] = jnp.transpose(
        v_ref[...].reshape(4 * HW, c2).astype(jnp.bfloat16))


def _super_weights(w, scale, bias):
    cin, cout = w.shape[2], w.shape[3]
    we = w * scale.reshape(1, 1, 1, cout)
    z = jnp.zeros((cin, cout), jnp.float32)
    left = jnp.concatenate(
        [jnp.concatenate([we[a, 0], we[a, 1], we[a, 2], z], axis=0)
         for a in range(3)], axis=0)
    right = jnp.concatenate(
        [jnp.concatenate([z, we[a, 0], we[a, 1], we[a, 2]], axis=0)
         for a in range(3)], axis=0)
    wsup = jnp.concatenate([left, right], axis=1).astype(jnp.bfloat16)
    b2 = jnp.concatenate([bias.reshape(1, cout)] * 2, axis=1)
    return wsup, b2


def kernel(x, indices, w0, scale0, bias0, w1, scale1, bias1):
    N, Cin, H, W = x.shape
    HW = H * W
    C1 = w0.shape[3]
    C2 = w1.shape[3]
    xr = x.astype(jnp.float32).reshape(N, Cin, HW)
    ir = indices.astype(jnp.int32).reshape(N, Cin, HW)
    ws0, b0 = _super_weights(w0, scale0, bias0)
    ws1, b1 = _super_weights(w1, scale1, bias1)

    out = pl.pallas_call(
        functools.partial(_decoder_body, H=H, W=W),
        out_shape=jax.ShapeDtypeStruct((N, C2, 4 * HW), jnp.bfloat16),
        grid=(N,),
        in_specs=[
            pl.BlockSpec((1, Cin, HW), lambda n: (n, 0, 0)),
            pl.BlockSpec((1, Cin, HW), lambda n: (n, 0, 0)),
            pl.BlockSpec(ws0.shape, lambda n: (0, 0)),
            pl.BlockSpec(b0.shape, lambda n: (0, 0)),
            pl.BlockSpec(ws1.shape, lambda n: (0, 0)),
            pl.BlockSpec(b1.shape, lambda n: (0, 0)),
        ],
        out_specs=pl.BlockSpec((1, C2, 4 * HW), lambda n: (n, 0, 0)),
        scratch_shapes=[
            pltpu.VMEM((H, 2, 2 * W, C2), jnp.float32),
            pltpu.VMEM((4, HW + 2 * (W + 1), Cin), jnp.bfloat16),
            pltpu.VMEM((4, HW + 2 * (W + 1), C1), jnp.bfloat16),
        ],
        compiler_params=pltpu.CompilerParams(
            dimension_semantics=("parallel",)),
    )(xr, ir, ws0, b0, ws1, b1)

    return out.reshape(N, C2, 2 * H, 2 * W).astype(jnp.float32)


# speedup vs baseline: 1.1401x; 1.1401x over previous
import functools

import jax
import jax.numpy as jnp
from jax import lax
from jax.experimental import pallas as pl
from jax.experimental.pallas import tpu as pltpu

_XTAPS = ((1, -1), (0, 0), (1, 0), (0, 1))


def _decoder_body(x_ref, i_ref, w0_ref, b0_ref, w1_ref, b1_ref, o_ref,
                  v_ref, *, H, W):
    HW = H * W
    P = W + 1
    xv = x_ref[0].astype(jnp.bfloat16)
    iv = i_ref[0]
    lane = lax.broadcasted_iota(jnp.int32, (1, HW), 1)
    ih = lane // W
    iw = lane - W * ih
    base = (2 * ih) * (2 * W) + 2 * iw

    phases = {}
    for py in range(2):
        for px in range(2):
            m = jnp.where(iv == base + (py * 2 * W + px), xv,
                          jnp.bfloat16(0))
            phases[(py, px)] = jnp.transpose(m)

    col_id = lax.broadcasted_iota(jnp.int32, (HW, 1), 0) % W
    not_left = col_id != 0
    not_right = col_id != (W - 1)

    def conv_layer(ph, wm, bi, out_f32):
        cin = ph[(0, 0)].shape[1]
        zpad = jnp.zeros((P, cin), jnp.bfloat16)
        ap = {k: jnp.concatenate([zpad, v, zpad], axis=0)
              for k, v in ph.items()}
        zs = []
        for py in range(2):
            taps = []
            for a in range(3):
                oy = a - 1
                sy = (py + oy) % 2
                gy = (py + oy) // 2
                for b in range(4):
                    sx, gx = _XTAPS[b]
                    s = P + gy * W + gx
                    t = ap[(sy, sx)][s:s + HW, :]
                    if gx == -1:
                        t = jnp.where(not_left, t, 0)
                    elif gx == 1:
                        t = jnp.where(not_right, t, 0)
                    taps.append(t)
            col = jnp.concatenate(taps, axis=1)
            z = jnp.dot(col, wm, preferred_element_type=jnp.float32)
            if not out_f32:
                z = z.astype(jnp.bfloat16)
            zs.append(jnp.maximum(z + bi.astype(z.dtype), 0))
        return zs

    z1 = conv_layer(phases, w0_ref[...], b0_ref[...], False)
    c1 = z1[0].shape[1] // 2
    ph1 = {(py, px): z1[py][:, px * c1:(px + 1) * c1]
           for py in range(2) for px in range(2)}
    z2 = conv_layer(ph1, w1_ref[...], b1_ref[...], True)
    c2 = z2[0].shape[1] // 2
    for py in range(2):
        for px in range(2):
            v_ref[:, py, pl.Slice(px, W, 2), :] = (
                z2[py][:, px * c2:(px + 1) * c2].reshape(H, W, c2))
    o_ref[0] = jnp.transpose(
        v_ref[...].reshape(4 * HW, c2).astype(jnp.bfloat16))


def _super_weights(w, scale, bias):
    cin, cout = w.shape[2], w.shape[3]
    we = w * scale.reshape(1, 1, 1, cout)
    z = jnp.zeros((cin, cout), jnp.float32)
    left = jnp.concatenate(
        [jnp.concatenate([we[a, 0], we[a, 1], we[a, 2], z], axis=0)
         for a in range(3)], axis=0)
    right = jnp.concatenate(
        [jnp.concatenate([z, we[a, 0], we[a, 1], we[a, 2]], axis=0)
         for a in range(3)], axis=0)
    wsup = jnp.concatenate([left, right], axis=1).astype(jnp.bfloat16)
    b2 = jnp.concatenate([bias.reshape(1, cout)] * 2, axis=1)
    return wsup, b2


def kernel(x, indices, w0, scale0, bias0, w1, scale1, bias1):
    N, Cin, H, W = x.shape
    HW = H * W
    C1 = w0.shape[3]
    C2 = w1.shape[3]
    xr = x.astype(jnp.float32).reshape(N, Cin, HW)
    ir = indices.astype(jnp.int32).reshape(N, Cin, HW)
    ws0, b0 = _super_weights(w0, scale0, bias0)
    ws1, b1 = _super_weights(w1, scale1, bias1)

    out = pl.pallas_call(
        functools.partial(_decoder_body, H=H, W=W),
        out_shape=jax.ShapeDtypeStruct((N, C2, 4 * HW), jnp.bfloat16),
        grid=(N,),
        in_specs=[
            pl.BlockSpec((1, Cin, HW), lambda n: (n, 0, 0)),
            pl.BlockSpec((1, Cin, HW), lambda n: (n, 0, 0)),
            pl.BlockSpec(ws0.shape, lambda n: (0, 0)),
            pl.BlockSpec(b0.shape, lambda n: (0, 0)),
            pl.BlockSpec(ws1.shape, lambda n: (0, 0)),
            pl.BlockSpec(b1.shape, lambda n: (0, 0)),
        ],
        out_specs=pl.BlockSpec((1, C2, 4 * HW), lambda n: (n, 0, 0)),
        scratch_shapes=[pltpu.VMEM((H, 2, 2 * W, C2), jnp.float32)],
        compiler_params=pltpu.CompilerParams(
            dimension_semantics=("parallel",)),
    )(xr, ir, ws0, b0, ws1, b1)

    return out.reshape(N, C2, 2 * H, 2 * W).astype(jnp.float32)
